# baseline (device time: 20179 ns/iter reference)
import jax
import jax.numpy as jnp
from jax import lax
from jax.experimental import pallas as pl
from jax.experimental.pallas import tpu as pltpu

N_DEV = 4


def kernel(x, router_W, route_idx, expert_W):
    n_tok, d_model = x.shape
    n_exp = router_W.shape[1]
    e_local, _, d_out = expert_W.shape

    def body(x_ref, rw_ref, idx_ref, ew_ref, out_ref, comm_ref, send_sems, recv_sems):
        my_pos = lax.axis_index("i")
        left = lax.rem(my_pos - 1 + N_DEV, N_DEV)
        right = lax.rem(my_pos + 1, N_DEV)

        barrier_sem = pltpu.get_barrier_semaphore()
        for nbr in [left, right]:
            pl.semaphore_signal(
                barrier_sem, inc=1,
                device_id=(nbr,), device_id_type=pl.DeviceIdType.MESH,
            )
        pl.semaphore_wait(barrier_sem, 2)

        xv = x_ref[:, :]

        scores = jnp.dot(xv, rw_ref[:, :], preferred_element_type=jnp.float32)
        s_max = jnp.max(scores, axis=-1, keepdims=True)
        p = jnp.exp(scores - s_max)
        probs = p / jnp.sum(p, axis=-1, keepdims=True)

        idx = idx_ref[:, :]
        e_iota = lax.broadcasted_iota(jnp.int32, (n_tok, n_exp), 1)
        sel = ((idx[:, 0:1] == e_iota) | (idx[:, 1:2] == e_iota)).astype(jnp.float32)
        denom = jnp.sum(probs * sel, axis=-1, keepdims=True)
        gates = probs * sel / denom

        partial = jnp.zeros((n_tok, d_out), jnp.float32)
        for j in range(e_local):
            gid = my_pos * e_local + j
            gate_j = jnp.sum(
                gates * (e_iota == gid).astype(jnp.float32),
                axis=-1, keepdims=True,
            )
            partial = partial + jnp.dot(
                xv * gate_j, ew_ref[j, :, :], preferred_element_type=jnp.float32
            )

        out_ref[:, :] = partial
        comm_ref[0, :, :] = partial

        for h in range(N_DEV - 1):
            send_slot = h % 2
            recv_slot = (h + 1) % 2
            rdma = pltpu.make_async_remote_copy(
                src_ref=comm_ref.at[send_slot],
                dst_ref=comm_ref.at[recv_slot],
                send_sem=send_sems.at[send_slot],
                recv_sem=recv_sems.at[recv_slot],
                device_id=(right,),
                device_id_type=pl.DeviceIdType.MESH,
            )
            rdma.start()
            rdma.wait()
            out_ref[:, :] = out_ref[:, :] + comm_ref[recv_slot, :, :]

    return pl.pallas_call(
        body,
        out_shape=jax.ShapeDtypeStruct((n_tok, d_out), jnp.float32),
        in_specs=[pl.BlockSpec(memory_space=pltpu.VMEM)] * 4,
        out_specs=pl.BlockSpec(memory_space=pltpu.VMEM),
        scratch_shapes=[
            pltpu.VMEM((2, n_tok, d_out), jnp.float32),
            pltpu.SemaphoreType.DMA((2,)),
            pltpu.SemaphoreType.DMA((2,)),
        ],
        compiler_params=pltpu.CompilerParams(collective_id=0),
    )(x, router_W, route_idx, expert_W)


# device time: 12528 ns/iter; 1.6107x vs baseline; 1.6107x over previous
import jax
import jax.numpy as jnp
from jax import lax
from jax.experimental import pallas as pl
from jax.experimental.pallas import tpu as pltpu

N_DEV = 4


def kernel(x, router_W, route_idx, expert_W):
    n_tok, d_model = x.shape
    n_exp = router_W.shape[1]
    e_local, _, d_out = expert_W.shape

    half = d_out // 2

    def body(x_ref, rw_ref, idx_ref, ew_ref, out_ref,
             accL_ref, accR_ref, comm_ref, send_sems, recv_sems):
        my_pos = lax.axis_index("i")
        pA = my_pos ^ 1
        pB = 3 - my_pos

        barrier_sem = pltpu.get_barrier_semaphore()
        for nbr in [pA, pB]:
            pl.semaphore_signal(
                barrier_sem, inc=1,
                device_id=(nbr,), device_id_type=pl.DeviceIdType.MESH,
            )
        pl.semaphore_wait(barrier_sem, 2)

        xv = x_ref[:, :]

        scores = jnp.dot(xv, rw_ref[:, :], preferred_element_type=jnp.float32)
        s_max = jnp.max(scores, axis=-1, keepdims=True)
        p = jnp.exp(scores - s_max)
        probs = p / jnp.sum(p, axis=-1, keepdims=True)

        idx = idx_ref[:, :]
        e_iota = lax.broadcasted_iota(jnp.int32, (n_tok, n_exp), 1)
        sel = ((idx[:, 0:1] == e_iota) | (idx[:, 1:2] == e_iota)).astype(jnp.float32)
        denom = jnp.sum(probs * sel, axis=-1, keepdims=True)
        gates = probs * sel / denom

        partial = jnp.zeros((n_tok, d_out), jnp.float32)
        for j in range(e_local):
            gid = my_pos * e_local + j
            gate_j = jnp.sum(
                gates * (e_iota == gid).astype(jnp.float32),
                axis=-1, keepdims=True,
            )
            partial = partial + jnp.dot(
                xv * gate_j, ew_ref[j, :, :], preferred_element_type=jnp.float32
            )

        accL_ref[:, :] = partial[:, :half]
        accR_ref[:, :] = partial[:, half:]

        def exchange(slot_l, slot_r, dev_l, dev_r):
            rl = pltpu.make_async_remote_copy(
                src_ref=accL_ref, dst_ref=comm_ref.at[slot_l],
                send_sem=send_sems.at[slot_l], recv_sem=recv_sems.at[slot_l],
                device_id=(dev_l,), device_id_type=pl.DeviceIdType.MESH,
            )
            rr = pltpu.make_async_remote_copy(
                src_ref=accR_ref, dst_ref=comm_ref.at[slot_r],
                send_sem=send_sems.at[slot_r], recv_sem=recv_sems.at[slot_r],
                device_id=(dev_r,), device_id_type=pl.DeviceIdType.MESH,
            )
            rl.start()
            rr.start()
            rl.wait()
            rr.wait()

        exchange(0, 1, pA, pB)
        accL_ref[:, :] = accL_ref[:, :] + comm_ref[0, :, :]
        accR_ref[:, :] = accR_ref[:, :] + comm_ref[1, :, :]

        exchange(2, 3, pB, pA)
        out_ref[:, :half] = accL_ref[:, :] + comm_ref[2, :, :]
        out_ref[:, half:] = accR_ref[:, :] + comm_ref[3, :, :]

    return pl.pallas_call(
        body,
        out_shape=jax.ShapeDtypeStruct((n_tok, d_out), jnp.float32),
        in_specs=[pl.BlockSpec(memory_space=pltpu.VMEM)] * 4,
        out_specs=pl.BlockSpec(memory_space=pltpu.VMEM),
        scratch_shapes=[
            pltpu.VMEM((n_tok, half), jnp.float32),
            pltpu.VMEM((n_tok, half), jnp.float32),
            pltpu.VMEM((4, n_tok, half), jnp.float32),
            pltpu.SemaphoreType.DMA((4,)),
            pltpu.SemaphoreType.DMA((4,)),
        ],
        compiler_params=pltpu.CompilerParams(collective_id=0),
    )(x, router_W, route_idx, expert_W)
